# baseline (speedup 1.0000x reference)
import jax
import jax.numpy as jnp
from jax import lax
from jax.experimental import pallas as pl

B = 1024
S = 26
C = 1000
ROW = S * C
BR = 64


def _body(batch_ref, gb_ref, m_ref, out_ref):
    idx = batch_ref[...]
    hi = (idx >> 3).astype(jnp.bfloat16)
    lo = (idx & 7).astype(jnp.bfloat16)
    a = jnp.concatenate([hi, lo], axis=1)
    t = jnp.dot(a, gb_ref[...], preferred_element_type=jnp.float32)
    m = m_ref[...]
    out_ref[...] = jnp.where(t == m, 1.0, 0.0)


@jax.jit
def _onehot_tc(batch):
    cols = jnp.arange(ROW, dtype=jnp.int32)
    g = (cols[None, :] // C == jnp.arange(S, dtype=jnp.int32)[:, None])
    g = g.astype(jnp.float32)
    gb = jnp.concatenate([8.0 * g, g], axis=0).astype(jnp.bfloat16)
    m = (cols % C).astype(jnp.float32)[None, :]
    return pl.pallas_call(
        _body,
        out_shape=jax.ShapeDtypeStruct((B, ROW), jnp.float32),
        grid=(B // BR,),
        in_specs=[
            pl.BlockSpec((BR, S), lambda i: (i, 0)),
            pl.BlockSpec((2 * S, ROW), lambda i: (0, 0)),
            pl.BlockSpec((1, ROW), lambda i: (0, 0)),
        ],
        out_specs=pl.BlockSpec((BR, ROW), lambda i: (i, 0)),
    )(batch, gb, m)


def kernel(batch, lookup):
    del lookup
    return _onehot_tc(jnp.asarray(batch, jnp.int32))
